# Initial kernel scaffold; baseline (speedup 1.0000x reference)
import jax
import jax.numpy as jnp
from jax.experimental import pallas as pl


def kernel(raw, edge_index, W_mlp, b_mlp, Wr0, Wn0, b0, Wr1, Wn1, b1, ln_g, ln_b, W_post, b_post, W_out, b_out):
    raise NotImplementedError("write your pallas kernel here")


# speedup vs baseline: 3.1724x; 3.1724x over previous
import functools

import jax
import jax.numpy as jnp
from jax import lax
from jax.experimental import pallas as pl
from jax.experimental.pallas import tpu as pltpu
from jax.experimental.pallas import tpu_sc as plsc

F32 = jnp.float32
_DN = (((1,), (1,)), ((), ()))


def _gelu(x):
    return 0.5 * x * (1.0 + lax.erf(x * 0.7071067811865476))


def _pre_body(raw_ref, wm_ref, bm_ref, wr_ref, wn_ref, b0_ref, u_ref, v_ref):
    x = _gelu(lax.dot_general(raw_ref[...], wm_ref[...], _DN,
                              preferred_element_type=F32) + bm_ref[...])
    u_ref[...] = lax.dot_general(x, wr_ref[...], _DN,
                                 preferred_element_type=F32) + b0_ref[...]
    v = lax.dot_general(x, wn_ref[...], _DN, preferred_element_type=F32)
    v_ref[0] = v[:, :128]
    v_ref[1] = v[:, 128:]


def _mid_body(u_ref, agg_ref, deg_ref, wr_ref, wn_ref, b_ref, u_ref_o, v_ref):
    agg = jnp.concatenate([agg_ref[0], agg_ref[1]], axis=-1)
    deg = jnp.maximum(jnp.sum(deg_ref[...], axis=-1, keepdims=True), 1.0)
    x = _gelu(u_ref[...] + agg / deg)
    u_ref_o[...] = lax.dot_general(x, wr_ref[...], _DN,
                                   preferred_element_type=F32) + b_ref[...]
    v = lax.dot_general(x, wn_ref[...], _DN, preferred_element_type=F32)
    v_ref[0] = v[:, :128]
    v_ref[1] = v[:, 128:]


def _post_body(raw_ref, u_ref, agg_ref, deg_ref, lng_ref, lnb_ref,
               wp_ref, bp_ref, wo_ref, bo_ref, out_ref, ne_ref):
    agg = jnp.concatenate([agg_ref[0], agg_ref[1]], axis=-1)
    deg = jnp.maximum(jnp.sum(deg_ref[...], axis=-1, keepdims=True), 1.0)
    ne = u_ref[...] + agg / deg
    ne_ref[...] = ne
    x = _gelu(ne)
    cc = jnp.concatenate([raw_ref[...], x], axis=-1)
    m = jnp.mean(cc, axis=-1, keepdims=True)
    v = jnp.mean((cc - m) ** 2, axis=-1, keepdims=True)
    y = (cc - m) / jnp.sqrt(v + 1e-5) * lng_ref[...] + lnb_ref[...]
    p = _gelu(lax.dot_general(y, wp_ref[...], _DN,
                              preferred_element_type=F32) + bp_ref[...])
    out_ref[...] = jax.nn.sigmoid(
        lax.dot_general(p, wo_ref[...], _DN,
                        preferred_element_type=F32) + bo_ref[...])


def _full(shape):
    return pl.BlockSpec(shape, lambda i: (0,) * len(shape))


def _rows(r, cols):
    return pl.BlockSpec((r, cols), lambda i: (i, 0))


def _split_rows(r):
    return pl.BlockSpec((2, r, 128), lambda i: (0, i, 0))


def _make_sc_agg(n_pad, epadr, with_deg):
    n_sub = 16
    cpt = epadr // n_sub
    rpt = n_pad // n_sub
    nz = rpt // 128
    ndr = n_pad // 128

    outs = [jax.ShapeDtypeStruct((2, n_pad, 128), F32)]
    scratch = [
        pltpu.VMEM_SHARED((n_pad, 128), F32),
        pltpu.VMEM((8, 128), jnp.int32),
        pltpu.VMEM((8, 128), jnp.int32),
        pltpu.VMEM((128, 128), F32),
        pltpu.SemaphoreType.DMA,
    ]
    if with_deg:
        outs.append(jax.ShapeDtypeStruct((n_sub, ndr, 128), F32))
        scratch.append(pltpu.VMEM((ndr, 128), F32))

    mesh = plsc.VectorSubcoreMesh(core_axis_name="c", subcore_axis_name="s")

    @functools.partial(
        pl.kernel, mesh=mesh, out_type=outs, scratch_types=scratch,
        compiler_params=pltpu.CompilerParams(needs_layout_passes=False))
    def k(tbl, spst, dpr, z128, zdeg, *rest):
        if with_deg:
            (out_agg, out_deg, agg_s, sidx, didx, rows, sem, deg_part) = rest
        else:
            (out_agg, agg_s, sidx, didx, rows, sem) = rest
        c = lax.axis_index("c")
        s = lax.axis_index("s")

        pltpu.sync_copy(z128, rows)
        for i in range(nz):
            pltpu.sync_copy(rows, agg_s.at[pl.ds(s * rpt + i * 128, 128)])
        if with_deg:
            pltpu.sync_copy(zdeg, deg_part)
        plsc.subcore_barrier()

        ones = jnp.full((16,), 1.0, F32)
        m127 = jnp.full((16,), 127, jnp.int32)

        def body(jb, carry):
            pltpu.sync_copy(spst.at[c, pl.ds(s * cpt + jb * 8, 8)], sidx)
            pltpu.sync_copy(dpr.at[pl.ds(s * cpt + jb * 8, 8)], didx)
            for j in range(8):
                pltpu.async_copy(tbl.at[sidx.at[j]], rows, sem).wait()
                pltpu.sync_copy(rows, agg_s.at[didx.at[j]], add=True)
                if with_deg:
                    for kk in range(8):
                        idxv = didx[j, pl.ds(kk * 16, 16)]
                        r = lax.shift_right_logical(idxv, 7)
                        q = lax.bitwise_and(idxv, m127)
                        plsc.addupdate_scatter(deg_part, [r, q], ones)
            return carry
        lax.fori_loop(0, cpt // 8, body, 0)

        plsc.subcore_barrier()

        pltpu.sync_copy(agg_s.at[pl.ds(s * rpt, rpt)],
                        out_agg.at[c, pl.ds(s * rpt, rpt)])
        if with_deg:
            @pl.when(c == 0)
            def _():
                pltpu.sync_copy(deg_part, out_deg.at[s])

    return k


def kernel(raw, edge_index, W_mlp, b_mlp, Wr0, Wn0, b0, Wr1, Wn1, b1,
           ln_g, ln_b, W_post, b_post, W_out, b_out):
    N, D = raw.shape
    H = W_mlp.shape[0]
    C = W_out.shape[0]
    E = edge_index.shape[1]

    EPAD = -(-E // 16384) * 16384
    NPAD = -(-(N + 1) // 2048) * 2048
    epadr = EPAD // 128

    src = edge_index[0]
    dst = edge_index[1]
    pad = EPAD - E
    sp = jnp.concatenate([src, jnp.zeros((pad,), jnp.int32)])
    dp = jnp.concatenate([dst, jnp.full((pad,), N, jnp.int32)])
    spr = sp.reshape(epadr, 128)
    spst = jnp.stack([spr, spr + N])
    dpr = dp.reshape(epadr, 128)
    z128 = jnp.zeros((128, 128), F32)
    zdeg = jnp.zeros((NPAD // 128, 128), F32)

    bm = b_mlp.reshape(1, H)
    b0r = b0.reshape(1, H)
    b1r = b1.reshape(1, H)
    lng = ln_g.reshape(1, D + H)
    lnb = ln_b.reshape(1, D + H)
    bp = b_post.reshape(1, H)
    bo = b_out.reshape(1, C)

    R = 1000
    grid = (N // R,)

    pre = pl.pallas_call(
        _pre_body, grid=grid,
        in_specs=[_rows(R, D), _full((H, D)), _full((1, H)),
                  _full((H, H)), _full((H, H)), _full((1, H))],
        out_specs=[_rows(R, H), _split_rows(R)],
        out_shape=[jax.ShapeDtypeStruct((N, H), F32),
                   jax.ShapeDtypeStruct((2, N, 128), F32)],
    )
    u0, v0s = pre(raw, W_mlp, bm, Wr0, Wn0, b0r)

    sc0 = _make_sc_agg(NPAD, epadr, with_deg=True)
    agg0, deg_p = sc0(v0s.reshape(2 * N, 128), spst, dpr, z128, zdeg)
    deg = jnp.transpose(deg_p.reshape(16, NPAD))

    mid = pl.pallas_call(
        _mid_body, grid=grid,
        in_specs=[_rows(R, H), _split_rows(R), _rows(R, 16),
                  _full((H, H)), _full((H, H)), _full((1, H))],
        out_specs=[_rows(R, H), _split_rows(R)],
        out_shape=[jax.ShapeDtypeStruct((N, H), F32),
                   jax.ShapeDtypeStruct((2, N, 128), F32)],
    )
    u1, v1s = mid(u0, agg0, deg, Wr1, Wn1, b1r)

    sc1 = _make_sc_agg(NPAD, epadr, with_deg=False)
    (agg1,) = sc1(v1s.reshape(2 * N, 128), spst, dpr, z128, zdeg)

    post = pl.pallas_call(
        _post_body, grid=grid,
        in_specs=[_rows(R, D), _rows(R, H), _split_rows(R), _rows(R, 16),
                  _full((1, D + H)), _full((1, D + H)),
                  _full((H, D + H)), _full((1, H)),
                  _full((C, H)), _full((1, C))],
        out_specs=[_rows(R, C), _rows(R, H)],
        out_shape=[jax.ShapeDtypeStruct((N, C), F32),
                   jax.ShapeDtypeStruct((N, H), F32)],
    )
    out, ne = post(raw, u1, agg1, deg, lng, lnb, W_post, bp, W_out, bo)
    return (out, ne)


# speedup vs baseline: 3.5483x; 1.1185x over previous
import functools

import jax
import jax.numpy as jnp
from jax import lax
from jax.experimental import pallas as pl
from jax.experimental.pallas import tpu as pltpu
from jax.experimental.pallas import tpu_sc as plsc

F32 = jnp.float32
_DN = (((1,), (1,)), ((), ()))


def _gelu(x):
    return 0.5 * x * (1.0 + lax.erf(x * 0.7071067811865476))


def _pre_body(raw_ref, wm_ref, bm_ref, wr_ref, wn_ref, b0_ref, u_ref, v_ref):
    x = _gelu(lax.dot_general(raw_ref[...], wm_ref[...], _DN,
                              preferred_element_type=F32) + bm_ref[...])
    u_ref[...] = lax.dot_general(x, wr_ref[...], _DN,
                                 preferred_element_type=F32) + b0_ref[...]
    v = lax.dot_general(x, wn_ref[...], _DN, preferred_element_type=F32)
    v_ref[0] = v[:, :128]
    v_ref[1] = v[:, 128:]


def _mid_body(u_ref, agg_ref, deg_ref, wr_ref, wn_ref, b_ref, u_ref_o, v_ref):
    agg = jnp.concatenate([agg_ref[0], agg_ref[1]], axis=-1)
    deg = jnp.maximum(jnp.sum(deg_ref[...], axis=-1, keepdims=True), 1.0)
    x = _gelu(u_ref[...] + agg / deg)
    u_ref_o[...] = lax.dot_general(x, wr_ref[...], _DN,
                                   preferred_element_type=F32) + b_ref[...]
    v = lax.dot_general(x, wn_ref[...], _DN, preferred_element_type=F32)
    v_ref[0] = v[:, :128]
    v_ref[1] = v[:, 128:]


def _post_body(raw_ref, u_ref, agg_ref, deg_ref, lng_ref, lnb_ref,
               wp_ref, bp_ref, wo_ref, bo_ref, out_ref, ne_ref):
    agg = jnp.concatenate([agg_ref[0], agg_ref[1]], axis=-1)
    deg = jnp.maximum(jnp.sum(deg_ref[...], axis=-1, keepdims=True), 1.0)
    ne = u_ref[...] + agg / deg
    ne_ref[...] = ne
    x = _gelu(ne)
    cc = jnp.concatenate([raw_ref[...], x], axis=-1)
    m = jnp.mean(cc, axis=-1, keepdims=True)
    v = jnp.mean((cc - m) ** 2, axis=-1, keepdims=True)
    y = (cc - m) / jnp.sqrt(v + 1e-5) * lng_ref[...] + lnb_ref[...]
    p = _gelu(lax.dot_general(y, wp_ref[...], _DN,
                              preferred_element_type=F32) + bp_ref[...])
    out_ref[...] = jax.nn.sigmoid(
        lax.dot_general(p, wo_ref[...], _DN,
                        preferred_element_type=F32) + bo_ref[...])


def _full(shape):
    return pl.BlockSpec(shape, lambda i: (0,) * len(shape))


def _rows(r, cols):
    return pl.BlockSpec((r, cols), lambda i: (i, 0))


def _split_rows(r):
    return pl.BlockSpec((2, r, 128), lambda i: (0, i, 0))


def _make_sc_agg(n_pad, epadr, with_deg):
    n_sub = 16
    cpt = epadr // n_sub
    rpt = n_pad // n_sub
    nz = rpt // 128
    ndr = n_pad // 128

    nblk = cpt // 8

    outs = [jax.ShapeDtypeStruct((2, n_pad, 128), F32)]
    scratch = [
        pltpu.VMEM_SHARED((n_pad, 128), F32),
        pltpu.VMEM((8, 128), jnp.int32),
        pltpu.VMEM((8, 128), jnp.int32),
        pltpu.VMEM((128, 128), F32),
        pltpu.VMEM((128, 128), F32),
        pltpu.SemaphoreType.DMA,
    ]
    if with_deg:
        outs.append(jax.ShapeDtypeStruct((n_sub, ndr, 128), F32))
        scratch.append(pltpu.VMEM((ndr, 128), F32))

    mesh = plsc.VectorSubcoreMesh(core_axis_name="c", subcore_axis_name="s")

    @functools.partial(
        pl.kernel, mesh=mesh, out_type=outs, scratch_types=scratch,
        compiler_params=pltpu.CompilerParams(needs_layout_passes=False))
    def k(tbl, spst, dpr, z128, zdeg, *rest):
        if with_deg:
            (out_agg, out_deg, agg_s, sidx, didx, rowsA, rowsB, sem,
             deg_part) = rest
        else:
            (out_agg, agg_s, sidx, didx, rowsA, rowsB, sem) = rest
        c = lax.axis_index("c")
        s = lax.axis_index("s")

        pltpu.sync_copy(z128, rowsA)
        for i in range(nz):
            pltpu.sync_copy(rowsA, agg_s.at[pl.ds(s * rpt + i * 128, 128)])
        if with_deg:
            pltpu.sync_copy(zdeg, deg_part)
        plsc.subcore_barrier()

        ones = jnp.full((16,), 1.0, F32)
        m127 = jnp.full((16,), 127, jnp.int32)
        bufs = [rowsA, rowsB]

        def scat_deg(j):
            if with_deg:
                for kk in range(8):
                    idxv = didx[j, pl.ds(kk * 16, 16)]
                    r = lax.shift_right_logical(idxv, 7)
                    q = lax.bitwise_and(idxv, m127)
                    plsc.addupdate_scatter(deg_part, [r, q], ones)

        pltpu.sync_copy(spst.at[c, pl.ds(s * cpt, 8)], sidx)
        pltpu.sync_copy(dpr.at[pl.ds(s * cpt, 8)], didx)
        pltpu.async_copy(tbl.at[sidx.at[0]], rowsA, sem)

        def body(jb, carry):
            for j in range(8):
                cur, nxt = bufs[j % 2], bufs[(j + 1) % 2]
                pltpu.make_async_copy(tbl.at[sidx.at[j]], cur, sem).wait()
                scat_deg(j)
                if j < 7:
                    pltpu.async_copy(tbl.at[sidx.at[j + 1]], nxt, sem)
                    pltpu.sync_copy(cur, agg_s.at[didx.at[j]], add=True)
                else:
                    pltpu.sync_copy(cur, agg_s.at[didx.at[j]], add=True)

                    @pl.when(jb < nblk - 1)
                    def _():
                        pltpu.sync_copy(
                            spst.at[c, pl.ds(s * cpt + (jb + 1) * 8, 8)], sidx)
                        pltpu.sync_copy(
                            dpr.at[pl.ds(s * cpt + (jb + 1) * 8, 8)], didx)
                        pltpu.async_copy(tbl.at[sidx.at[0]], nxt, sem)
            return carry
        lax.fori_loop(0, nblk, body, 0)

        plsc.subcore_barrier()

        pltpu.sync_copy(agg_s.at[pl.ds(s * rpt, rpt)],
                        out_agg.at[c, pl.ds(s * rpt, rpt)])
        if with_deg:
            @pl.when(c == 0)
            def _():
                pltpu.sync_copy(deg_part, out_deg.at[s])

    return k


def kernel(raw, edge_index, W_mlp, b_mlp, Wr0, Wn0, b0, Wr1, Wn1, b1,
           ln_g, ln_b, W_post, b_post, W_out, b_out):
    N, D = raw.shape
    H = W_mlp.shape[0]
    C = W_out.shape[0]
    E = edge_index.shape[1]

    EPAD = -(-E // 16384) * 16384
    NPAD = -(-(N + 1) // 2048) * 2048
    epadr = EPAD // 128

    src = edge_index[0]
    dst = edge_index[1]
    pad = EPAD - E
    sp = jnp.concatenate([src, jnp.zeros((pad,), jnp.int32)])
    dp = jnp.concatenate([dst, jnp.full((pad,), N, jnp.int32)])
    spr = sp.reshape(epadr, 128)
    spst = jnp.stack([spr, spr + N])
    dpr = dp.reshape(epadr, 128)
    z128 = jnp.zeros((128, 128), F32)
    zdeg = jnp.zeros((NPAD // 128, 128), F32)

    bm = b_mlp.reshape(1, H)
    b0r = b0.reshape(1, H)
    b1r = b1.reshape(1, H)
    lng = ln_g.reshape(1, D + H)
    lnb = ln_b.reshape(1, D + H)
    bp = b_post.reshape(1, H)
    bo = b_out.reshape(1, C)

    R = 1000
    grid = (N // R,)

    pre = pl.pallas_call(
        _pre_body, grid=grid,
        in_specs=[_rows(R, D), _full((H, D)), _full((1, H)),
                  _full((H, H)), _full((H, H)), _full((1, H))],
        out_specs=[_rows(R, H), _split_rows(R)],
        out_shape=[jax.ShapeDtypeStruct((N, H), F32),
                   jax.ShapeDtypeStruct((2, N, 128), F32)],
    )
    u0, v0s = pre(raw, W_mlp, bm, Wr0, Wn0, b0r)

    sc0 = _make_sc_agg(NPAD, epadr, with_deg=True)
    agg0, deg_p = sc0(v0s.reshape(2 * N, 128), spst, dpr, z128, zdeg)
    deg = jnp.transpose(deg_p.reshape(16, NPAD))

    mid = pl.pallas_call(
        _mid_body, grid=grid,
        in_specs=[_rows(R, H), _split_rows(R), _rows(R, 16),
                  _full((H, H)), _full((H, H)), _full((1, H))],
        out_specs=[_rows(R, H), _split_rows(R)],
        out_shape=[jax.ShapeDtypeStruct((N, H), F32),
                   jax.ShapeDtypeStruct((2, N, 128), F32)],
    )
    u1, v1s = mid(u0, agg0, deg, Wr1, Wn1, b1r)

    sc1 = _make_sc_agg(NPAD, epadr, with_deg=False)
    (agg1,) = sc1(v1s.reshape(2 * N, 128), spst, dpr, z128, zdeg)

    post = pl.pallas_call(
        _post_body, grid=grid,
        in_specs=[_rows(R, D), _rows(R, H), _split_rows(R), _rows(R, 16),
                  _full((1, D + H)), _full((1, D + H)),
                  _full((H, D + H)), _full((1, H)),
                  _full((C, H)), _full((1, C))],
        out_specs=[_rows(R, C), _rows(R, H)],
        out_shape=[jax.ShapeDtypeStruct((N, C), F32),
                   jax.ShapeDtypeStruct((N, H), F32)],
    )
    out, ne = post(raw, u1, agg1, deg, lng, lnb, W_post, bp, W_out, bo)
    return (out, ne)


# speedup vs baseline: 3.5530x; 1.0013x over previous
import functools

import jax
import jax.numpy as jnp
from jax import lax
from jax.experimental import pallas as pl
from jax.experimental.pallas import tpu as pltpu
from jax.experimental.pallas import tpu_sc as plsc

F32 = jnp.float32
_DN = (((1,), (1,)), ((), ()))


def _gelu(x):
    return 0.5 * x * (1.0 + lax.erf(x * 0.7071067811865476))


def _pre_body(raw_ref, wm_ref, bm_ref, wr_ref, wn_ref, b0_ref, u_ref, v_ref):
    x = _gelu(lax.dot_general(raw_ref[...], wm_ref[...], _DN,
                              preferred_element_type=F32) + bm_ref[...])
    u_ref[...] = lax.dot_general(x, wr_ref[...], _DN,
                                 preferred_element_type=F32) + b0_ref[...]
    v = lax.dot_general(x, wn_ref[...], _DN, preferred_element_type=F32)
    v_ref[0] = v[:, :128]
    v_ref[1] = v[:, 128:]


def _mid_body(u_ref, agg_ref, deg_ref, wr_ref, wn_ref, b_ref, u_ref_o, v_ref):
    agg = jnp.concatenate([agg_ref[0], agg_ref[1]], axis=-1)
    deg = jnp.maximum(jnp.sum(deg_ref[...], axis=-1, keepdims=True), 1.0)
    x = _gelu(u_ref[...] + agg / deg)
    u_ref_o[...] = lax.dot_general(x, wr_ref[...], _DN,
                                   preferred_element_type=F32) + b_ref[...]
    v = lax.dot_general(x, wn_ref[...], _DN, preferred_element_type=F32)
    v_ref[0] = v[:, :128]
    v_ref[1] = v[:, 128:]


def _post_body(raw_ref, u_ref, agg_ref, deg_ref, lng_ref, lnb_ref,
               wp_ref, bp_ref, wo_ref, bo_ref, out_ref, ne_ref):
    agg = jnp.concatenate([agg_ref[0], agg_ref[1]], axis=-1)
    deg = jnp.maximum(jnp.sum(deg_ref[...], axis=-1, keepdims=True), 1.0)
    ne = u_ref[...] + agg / deg
    ne_ref[...] = ne
    x = _gelu(ne)
    cc = jnp.concatenate([raw_ref[...], x], axis=-1)
    m = jnp.mean(cc, axis=-1, keepdims=True)
    v = jnp.mean((cc - m) ** 2, axis=-1, keepdims=True)
    y = (cc - m) / jnp.sqrt(v + 1e-5) * lng_ref[...] + lnb_ref[...]
    p = _gelu(lax.dot_general(y, wp_ref[...], _DN,
                              preferred_element_type=F32) + bp_ref[...])
    out_ref[...] = jax.nn.sigmoid(
        lax.dot_general(p, wo_ref[...], _DN,
                        preferred_element_type=F32) + bo_ref[...])


def _full(shape):
    return pl.BlockSpec(shape, lambda i: (0,) * len(shape))


def _rows(r, cols):
    return pl.BlockSpec((r, cols), lambda i: (i, 0))


def _split_rows(r):
    return pl.BlockSpec((2, r, 128), lambda i: (0, i, 0))


def _make_sc_agg(n_pad, epadr, with_deg):
    n_sub = 16
    cpt = epadr // n_sub
    rpt = n_pad // n_sub
    nz = rpt // 128
    ndr = n_pad // 128

    nblk = cpt // 8

    outs = [jax.ShapeDtypeStruct((2, n_pad, 128), F32)]
    scratch = [
        pltpu.VMEM_SHARED((n_pad, 128), F32),
        pltpu.VMEM((8, 128), jnp.int32),
        pltpu.VMEM((8, 128), jnp.int32),
        pltpu.VMEM((128, 128), F32),
        pltpu.VMEM((128, 128), F32),
        pltpu.SemaphoreType.DMA,
        pltpu.SemaphoreType.DMA,
    ]
    if with_deg:
        outs.append(jax.ShapeDtypeStruct((n_sub, ndr, 128), F32))
        scratch.append(pltpu.VMEM((ndr, 128), F32))

    mesh = plsc.VectorSubcoreMesh(core_axis_name="c", subcore_axis_name="s")

    @functools.partial(
        pl.kernel, mesh=mesh, out_type=outs, scratch_types=scratch,
        compiler_params=pltpu.CompilerParams(needs_layout_passes=False))
    def k(tbl, spst, dpr, z128, zdeg, *rest):
        if with_deg:
            (out_agg, out_deg, agg_s, sidx, didx, rowsA, rowsB, gsem, ssem,
             deg_part) = rest
        else:
            (out_agg, agg_s, sidx, didx, rowsA, rowsB, gsem, ssem) = rest
        c = lax.axis_index("c")
        s = lax.axis_index("s")

        pltpu.sync_copy(z128, rowsA)
        for i in range(nz):
            pltpu.sync_copy(rowsA, agg_s.at[pl.ds(s * rpt + i * 128, 128)])
        if with_deg:
            pltpu.sync_copy(zdeg, deg_part)
        plsc.subcore_barrier()

        ones = jnp.full((16,), 1.0, F32)
        m127 = jnp.full((16,), 127, jnp.int32)
        bufs = [rowsA, rowsB]

        def scat_deg(j):
            if with_deg:
                for kk in range(8):
                    idxv = didx[j, pl.ds(kk * 16, 16)]
                    r = lax.shift_right_logical(idxv, 7)
                    q = lax.bitwise_and(idxv, m127)
                    plsc.addupdate_scatter(deg_part, [r, q], ones)

        pltpu.sync_copy(spst.at[c, pl.ds(s * cpt, 8)], sidx)
        pltpu.sync_copy(dpr.at[pl.ds(s * cpt, 8)], didx)
        pltpu.async_copy(tbl.at[sidx.at[0]], rowsA, gsem)

        def body(jb, carry):
            sh = [None] * 8
            for j in range(8):
                cur, nxt = bufs[j % 2], bufs[(j + 1) % 2]
                pltpu.make_async_copy(tbl.at[sidx.at[j]], cur, gsem).wait()
                scat_deg(j)
                sh[j] = pltpu.async_copy(cur, agg_s.at[didx.at[j]], ssem,
                                         add=True)
                if j < 7:
                    if j >= 1:
                        sh[j - 1].wait()
                    pltpu.async_copy(tbl.at[sidx.at[j + 1]], nxt, gsem)
                else:
                    sh[6].wait()
                    sh[7].wait()

                    @pl.when(jb < nblk - 1)
                    def _():
                        pltpu.sync_copy(
                            spst.at[c, pl.ds(s * cpt + (jb + 1) * 8, 8)], sidx)
                        pltpu.sync_copy(
                            dpr.at[pl.ds(s * cpt + (jb + 1) * 8, 8)], didx)
                        pltpu.async_copy(tbl.at[sidx.at[0]], nxt, gsem)
            return carry
        lax.fori_loop(0, nblk, body, 0)

        plsc.subcore_barrier()

        pltpu.sync_copy(agg_s.at[pl.ds(s * rpt, rpt)],
                        out_agg.at[c, pl.ds(s * rpt, rpt)])
        if with_deg:
            @pl.when(c == 0)
            def _():
                pltpu.sync_copy(deg_part, out_deg.at[s])

    return k


def kernel(raw, edge_index, W_mlp, b_mlp, Wr0, Wn0, b0, Wr1, Wn1, b1,
           ln_g, ln_b, W_post, b_post, W_out, b_out):
    N, D = raw.shape
    H = W_mlp.shape[0]
    C = W_out.shape[0]
    E = edge_index.shape[1]

    EPAD = -(-E // 16384) * 16384
    NPAD = -(-(N + 1) // 2048) * 2048
    epadr = EPAD // 128

    src = edge_index[0]
    dst = edge_index[1]
    pad = EPAD - E
    sp = jnp.concatenate([src, jnp.zeros((pad,), jnp.int32)])
    dp = jnp.concatenate([dst, jnp.full((pad,), N, jnp.int32)])
    spr = sp.reshape(epadr, 128)
    spst = jnp.stack([spr, spr + N])
    dpr = dp.reshape(epadr, 128)
    z128 = jnp.zeros((128, 128), F32)
    zdeg = jnp.zeros((NPAD // 128, 128), F32)

    bm = b_mlp.reshape(1, H)
    b0r = b0.reshape(1, H)
    b1r = b1.reshape(1, H)
    lng = ln_g.reshape(1, D + H)
    lnb = ln_b.reshape(1, D + H)
    bp = b_post.reshape(1, H)
    bo = b_out.reshape(1, C)

    R = 1000
    grid = (N // R,)

    pre = pl.pallas_call(
        _pre_body, grid=grid,
        in_specs=[_rows(R, D), _full((H, D)), _full((1, H)),
                  _full((H, H)), _full((H, H)), _full((1, H))],
        out_specs=[_rows(R, H), _split_rows(R)],
        out_shape=[jax.ShapeDtypeStruct((N, H), F32),
                   jax.ShapeDtypeStruct((2, N, 128), F32)],
    )
    u0, v0s = pre(raw, W_mlp, bm, Wr0, Wn0, b0r)

    sc0 = _make_sc_agg(NPAD, epadr, with_deg=True)
    agg0, deg_p = sc0(v0s.reshape(2 * N, 128), spst, dpr, z128, zdeg)
    deg = jnp.transpose(deg_p.reshape(16, NPAD))

    mid = pl.pallas_call(
        _mid_body, grid=grid,
        in_specs=[_rows(R, H), _split_rows(R), _rows(R, 16),
                  _full((H, H)), _full((H, H)), _full((1, H))],
        out_specs=[_rows(R, H), _split_rows(R)],
        out_shape=[jax.ShapeDtypeStruct((N, H), F32),
                   jax.ShapeDtypeStruct((2, N, 128), F32)],
    )
    u1, v1s = mid(u0, agg0, deg, Wr1, Wn1, b1r)

    sc1 = _make_sc_agg(NPAD, epadr, with_deg=False)
    (agg1,) = sc1(v1s.reshape(2 * N, 128), spst, dpr, z128, zdeg)

    post = pl.pallas_call(
        _post_body, grid=grid,
        in_specs=[_rows(R, D), _rows(R, H), _split_rows(R), _rows(R, 16),
                  _full((1, D + H)), _full((1, D + H)),
                  _full((H, D + H)), _full((1, H)),
                  _full((C, H)), _full((1, C))],
        out_specs=[_rows(R, C), _rows(R, H)],
        out_shape=[jax.ShapeDtypeStruct((N, C), F32),
                   jax.ShapeDtypeStruct((N, H), F32)],
    )
    out, ne = post(raw, u1, agg1, deg, lng, lnb, W_post, bp, W_out, bo)
    return (out, ne)
